# baseline (speedup 1.0000x reference)
import functools

import jax
import jax.numpy as jnp
from jax import lax
from jax.experimental import pallas as pl
from jax.experimental.pallas import tpu as pltpu
from jax.experimental.pallas import tpu_sc as plsc

B, T = 4, 4096
N = B * T
DH = 64
DW = 2 * DH
HIDDEN = 1024
H1 = HIDDEN // 2
EPS = 1e-05

NC, NS = 2, 16
NW = NC * NS
BPW = N // NW
CH = 128
NCH = BPW // CH
NPASS = 2
SUB = BPW // NPASS
NCH_SUB = SUB // CH

TC_COLS = 8192

R = 512
NBLK = N // R


def _transpose_body(in_ref, out_ref):
  x = in_ref[...]
  r = lax.broadcasted_iota(jnp.int32, (DH, DH), 0)
  c = lax.broadcasted_iota(jnp.int32, (DH, DH), 1)
  eye = (r == c).astype(jnp.float32)
  out_ref[...] = lax.dot_general(
      x, eye, dimension_numbers=(((0,), (0,)), ((), ())),
      preferred_element_type=jnp.float32)


@jax.jit
def _transpose(tab_t):
  V = tab_t.shape[1]
  grid = (V + TC_COLS - 1) // TC_COLS
  return pl.pallas_call(
      _transpose_body,
      grid=(grid,),
      in_specs=[pl.BlockSpec((DH, TC_COLS), lambda i: (0, i))],
      out_specs=pl.BlockSpec((TC_COLS, DH), lambda i: (i, 0)),
      out_shape=jax.ShapeDtypeStruct((V, DH), jnp.float32),
      compiler_params=pltpu.CompilerParams(
          vmem_limit_bytes=100 * 1024 * 1024),
  )(tab_t)


def _sc_gather_body(fids_hbm, tids_hbm, ftab_hbm, ttab_hbm,
                    fout_hbm, tout_hbm,
                    fidx_v, tidx_v, frows_v, trows_v, fsem, tsem):
  wid = lax.axis_index("s") * NC + lax.axis_index("c")
  base = wid * BPW
  pltpu.sync_copy(fids_hbm.at[pl.ds(wid * NCH, NCH)], fidx_v)
  pltpu.sync_copy(tids_hbm.at[pl.ds(wid * NCH, NCH)], tidx_v)
  for p in range(NPASS):
    handles = []
    for j in range(NCH_SUB):
      c = p * NCH_SUB + j
      handles.append(pltpu.async_copy(
          ftab_hbm.at[fidx_v.at[c]], frows_v.at[pl.ds(j * CH, CH)], fsem))
      handles.append(pltpu.async_copy(
          ttab_hbm.at[tidx_v.at[c]], trows_v.at[pl.ds(j * CH, CH)], tsem))
    for h in handles:
      h.wait()
    pltpu.sync_copy(frows_v, fout_hbm.at[pl.ds(base + p * SUB, SUB)])
    pltpu.sync_copy(trows_v, tout_hbm.at[pl.ds(base + p * SUB, SUB)])


@jax.jit
def _sc_gather(fidx, tidx, func_table2, token_table2):
  mesh = plsc.VectorSubcoreMesh(
      core_axis_name="c", subcore_axis_name="s",
      num_cores=NC, num_subcores=NS)
  return pl.kernel(
      _sc_gather_body,
      out_type=[
          jax.ShapeDtypeStruct((N, DW), jnp.float32),
          jax.ShapeDtypeStruct((N, DW), jnp.float32),
      ],
      mesh=mesh,
      scratch_types=[
          pltpu.VMEM((NCH, CH), jnp.int32),
          pltpu.VMEM((NCH, CH), jnp.int32),
          pltpu.VMEM((SUB, DW), jnp.float32),
          pltpu.VMEM((SUB, DW), jnp.float32),
          pltpu.SemaphoreType.DMA,
          pltpu.SemaphoreType.DMA,
      ],
  )(fidx.reshape(N // CH, CH), tidx.reshape(N // CH, CH),
    func_table2, token_table2)


def _mlp_body(node_ref, pidx_ref, etype_ref, fpar_ref, tpar_ref,
              femb_ref, temb_ref,
              wn_ref, bn_ref, ot_ref, et_ref,
              w1_ref, b1_ref, w2_ref, b2_ref, w3_ref, b3_ref,
              g_ref, beta_ref, out_ref):
  f32 = jnp.float32
  nf = jnp.dot(node_ref[...], wn_ref[...], preferred_element_type=f32)
  nf = nf + bn_ref[...]
  o = (pidx_ref[:, 0:1] == pidx_ref[:, 1:2]).astype(f32)
  oe = ot_ref[0:1, :] * (1.0 - o) + ot_ref[1:2, :] * o
  eids = etype_ref[...]
  eoh = (eids == lax.broadcasted_iota(jnp.int32, (1, 4), 1)).astype(f32)
  ee = jnp.dot(eoh, et_ref[...], preferred_element_type=f32)
  fp = fpar_ref[...] > 0
  fe = jnp.where(fp, femb_ref[:, DH:DW], femb_ref[:, 0:DH])
  tp = tpar_ref[...] > 0
  te = jnp.where(tp, temb_ref[:, DH:DW], temb_ref[:, 0:DH])
  acc = jnp.dot(nf, w1_ref[0:DH, :], preferred_element_type=f32)
  acc += jnp.dot(oe, w1_ref[DH:2 * DH, :], preferred_element_type=f32)
  acc += jnp.dot(ee, w1_ref[2 * DH:3 * DH, :], preferred_element_type=f32)
  acc += jnp.dot(fe, w1_ref[3 * DH:4 * DH, :], preferred_element_type=f32)
  acc += jnp.dot(te, w1_ref[4 * DH:5 * DH, :], preferred_element_type=f32)
  acc += b1_ref[...]
  h = jnp.where(acc > 0, acc, 0.01 * acc)
  h = jnp.dot(h, w2_ref[...], preferred_element_type=f32) + b2_ref[...]
  h = jnp.where(h > 0, h, 0.01 * h)
  h = jnp.dot(h, w3_ref[...], preferred_element_type=f32) + b3_ref[...]
  mu = jnp.mean(h, axis=-1, keepdims=True)
  d = h - mu
  var = jnp.mean(d * d, axis=-1, keepdims=True)
  out_ref[...] = d * lax.rsqrt(var + EPS) * g_ref[...] + beta_ref[...]


@jax.jit
def _mlp(node_data, padded_index, etype_ids, fpar, tpar, femb, temb,
         W_node, b_node, order_table, etype_table,
         W1, b1, W2, b2, W3, b3, ln_gamma, ln_beta):
  row = lambda i: (i, 0)
  const = lambda i: (0, 0)
  return pl.pallas_call(
      _mlp_body,
      grid=(NBLK,),
      in_specs=[
          pl.BlockSpec((R, 4), row),
          pl.BlockSpec((R, 2), row),
          pl.BlockSpec((R, 1), row),
          pl.BlockSpec((R, 1), row),
          pl.BlockSpec((R, 1), row),
          pl.BlockSpec((R, DW), row),
          pl.BlockSpec((R, DW), row),
          pl.BlockSpec((4, DH), const),
          pl.BlockSpec((1, DH), const),
          pl.BlockSpec((2, DH), const),
          pl.BlockSpec((4, DH), const),
          pl.BlockSpec((5 * DH, H1), const),
          pl.BlockSpec((1, H1), const),
          pl.BlockSpec((H1, HIDDEN), const),
          pl.BlockSpec((1, HIDDEN), const),
          pl.BlockSpec((HIDDEN, HIDDEN), const),
          pl.BlockSpec((1, HIDDEN), const),
          pl.BlockSpec((1, HIDDEN), const),
          pl.BlockSpec((1, HIDDEN), const),
      ],
      out_specs=pl.BlockSpec((R, HIDDEN), row),
      out_shape=jax.ShapeDtypeStruct((N, HIDDEN), jnp.float32),
  )(node_data, padded_index, etype_ids, fpar, tpar, femb, temb,
    W_node, b_node, order_table, etype_table,
    W1, b1, W2, b2, W3, b3, ln_gamma, ln_beta)


def kernel(node_data, padded_index, etype_ids, func_ids, token_ids,
           W_node, b_node, order_table, etype_table, func_table, token_table,
           W1, b1, W2, b2, W3, b3, ln_gamma, ln_beta):
  fids = func_ids.reshape(N)
  tids = token_ids.reshape(N)
  ftab_rm = _transpose(func_table.T)
  ttab_rm = _transpose(token_table.T)
  femb, temb = _sc_gather(
      fids >> 1, tids >> 1,
      ftab_rm.reshape(ftab_rm.shape[0] // 2, DW),
      ttab_rm.reshape(ttab_rm.shape[0] // 2, DW))
  out = _mlp(node_data.reshape(N, 4), padded_index.reshape(N, 2),
             etype_ids.reshape(N, 1), (fids & 1).reshape(N, 1),
             (tids & 1).reshape(N, 1), femb, temb,
             W_node, b_node.reshape(1, DH), order_table, etype_table,
             W1, b1.reshape(1, H1), W2, b2.reshape(1, HIDDEN),
             W3, b3.reshape(1, HIDDEN), ln_gamma.reshape(1, HIDDEN),
             ln_beta.reshape(1, HIDDEN))
  return out.reshape(B, T, HIDDEN)


# speedup vs baseline: 1.1652x; 1.1652x over previous
import functools

import jax
import jax.numpy as jnp
from jax import lax
from jax.experimental import pallas as pl
from jax.experimental.pallas import tpu as pltpu
from jax.experimental.pallas import tpu_sc as plsc

B, T = 4, 4096
N = B * T
DH = 64
DW = 2 * DH
HIDDEN = 1024
H1 = HIDDEN // 2
EPS = 1e-05

NC, NS = 2, 16
NW = NC * NS
BPW = N // NW
CH = 128
NCH = BPW // CH
NPASS = 2
SUB = BPW // NPASS
NCH_SUB = SUB // CH

R = 512
NBLK = N // R

TC_COLS = 16384


def _transpose_body(in_ref, out_ref):
  x = in_ref[...]
  r = lax.broadcasted_iota(jnp.int32, (DH, DH), 0)
  c = lax.broadcasted_iota(jnp.int32, (DH, DH), 1)
  eye = (r == c).astype(jnp.float32)
  out_ref[...] = lax.dot_general(
      x, eye, dimension_numbers=(((0,), (0,)), ((), ())),
      preferred_element_type=jnp.float32)


@jax.jit
def _transpose(tab_t):
  V = tab_t.shape[1]
  grid = (V + TC_COLS - 1) // TC_COLS
  return pl.pallas_call(
      _transpose_body,
      grid=(grid,),
      in_specs=[pl.BlockSpec((DH, TC_COLS), lambda i: (0, i))],
      out_specs=pl.BlockSpec((TC_COLS, DH), lambda i: (i, 0)),
      out_shape=jax.ShapeDtypeStruct((V, DH), jnp.float32),
  )(tab_t)


def _sc_gather_body(fids_hbm, tids_hbm, ftab_hbm, ttab_hbm,
                    fout_hbm, tout_hbm,
                    fidx_v, tidx_v, frows_v, trows_v, fsem, tsem):
  wid = lax.axis_index("s") * NC + lax.axis_index("c")
  base = wid * BPW
  pltpu.sync_copy(fids_hbm.at[pl.ds(wid * NCH, NCH)], fidx_v)
  pltpu.sync_copy(tids_hbm.at[pl.ds(wid * NCH, NCH)], tidx_v)
  for p in range(NPASS):
    handles = []
    for j in range(NCH_SUB):
      c = p * NCH_SUB + j
      handles.append(pltpu.async_copy(
          ftab_hbm.at[fidx_v.at[c]], frows_v.at[pl.ds(j * CH, CH)], fsem))
      handles.append(pltpu.async_copy(
          ttab_hbm.at[tidx_v.at[c]], trows_v.at[pl.ds(j * CH, CH)], tsem))
    for h in handles:
      h.wait()
    pltpu.sync_copy(frows_v, fout_hbm.at[pl.ds(base + p * SUB, SUB)])
    pltpu.sync_copy(trows_v, tout_hbm.at[pl.ds(base + p * SUB, SUB)])


@jax.jit
def _sc_gather(fidx, tidx, func_table2, token_table2):
  mesh = plsc.VectorSubcoreMesh(
      core_axis_name="c", subcore_axis_name="s",
      num_cores=NC, num_subcores=NS)
  return pl.kernel(
      _sc_gather_body,
      out_type=[
          jax.ShapeDtypeStruct((N, DW), jnp.float32),
          jax.ShapeDtypeStruct((N, DW), jnp.float32),
      ],
      mesh=mesh,
      scratch_types=[
          pltpu.VMEM((NCH, CH), jnp.int32),
          pltpu.VMEM((NCH, CH), jnp.int32),
          pltpu.VMEM((SUB, DW), jnp.float32),
          pltpu.VMEM((SUB, DW), jnp.float32),
          pltpu.SemaphoreType.DMA,
          pltpu.SemaphoreType.DMA,
      ],
  )(fidx.reshape(N // CH, CH), tidx.reshape(N // CH, CH),
    func_table2, token_table2)


def _mlp_body(node_ref, pidx_ref, etype_ref, fpar_ref, tpar_ref,
              femb_ref, temb_ref,
              wn_ref, bn_ref, ot_ref, et_ref,
              w1_ref, b1_ref, w2_ref, b2_ref, w3_ref, b3_ref,
              g_ref, beta_ref, out_ref):
  f32 = jnp.float32
  nf = jnp.dot(node_ref[...], wn_ref[...], preferred_element_type=f32)
  nf = nf + bn_ref[...]
  o = (pidx_ref[:, 0:1] == pidx_ref[:, 1:2]).astype(f32)
  oe = ot_ref[0:1, :] * (1.0 - o) + ot_ref[1:2, :] * o
  eids = etype_ref[...]
  eoh = (eids == lax.broadcasted_iota(jnp.int32, (1, 4), 1)).astype(f32)
  ee = jnp.dot(eoh, et_ref[...], preferred_element_type=f32)
  fp = fpar_ref[...] > 0
  fe = jnp.where(fp, femb_ref[:, DH:DW], femb_ref[:, 0:DH])
  tp = tpar_ref[...] > 0
  te = jnp.where(tp, temb_ref[:, DH:DW], temb_ref[:, 0:DH])
  acc = jnp.dot(nf, w1_ref[0:DH, :], preferred_element_type=f32)
  acc += jnp.dot(oe, w1_ref[DH:2 * DH, :], preferred_element_type=f32)
  acc += jnp.dot(ee, w1_ref[2 * DH:3 * DH, :], preferred_element_type=f32)
  acc += jnp.dot(fe, w1_ref[3 * DH:4 * DH, :], preferred_element_type=f32)
  acc += jnp.dot(te, w1_ref[4 * DH:5 * DH, :], preferred_element_type=f32)
  acc += b1_ref[...]
  h = jnp.where(acc > 0, acc, 0.01 * acc)
  h = jnp.dot(h, w2_ref[...], preferred_element_type=f32) + b2_ref[...]
  h = jnp.where(h > 0, h, 0.01 * h)
  h = jnp.dot(h, w3_ref[...], preferred_element_type=f32) + b3_ref[...]
  mu = jnp.mean(h, axis=-1, keepdims=True)
  d = h - mu
  var = jnp.mean(d * d, axis=-1, keepdims=True)
  out_ref[...] = d * lax.rsqrt(var + EPS) * g_ref[...] + beta_ref[...]


@jax.jit
def _mlp(node_data, padded_index, etype_ids, fpar, tpar, femb, temb,
         W_node, b_node, order_table, etype_table,
         W1, b1, W2, b2, W3, b3, ln_gamma, ln_beta):
  row = lambda i: (i, 0)
  const = lambda i: (0, 0)
  return pl.pallas_call(
      _mlp_body,
      grid=(NBLK,),
      in_specs=[
          pl.BlockSpec((R, 4), row),
          pl.BlockSpec((R, 2), row),
          pl.BlockSpec((R, 1), row),
          pl.BlockSpec((R, 1), row),
          pl.BlockSpec((R, 1), row),
          pl.BlockSpec((R, DW), row),
          pl.BlockSpec((R, DW), row),
          pl.BlockSpec((4, DH), const),
          pl.BlockSpec((1, DH), const),
          pl.BlockSpec((2, DH), const),
          pl.BlockSpec((4, DH), const),
          pl.BlockSpec((5 * DH, H1), const),
          pl.BlockSpec((1, H1), const),
          pl.BlockSpec((H1, HIDDEN), const),
          pl.BlockSpec((1, HIDDEN), const),
          pl.BlockSpec((HIDDEN, HIDDEN), const),
          pl.BlockSpec((1, HIDDEN), const),
          pl.BlockSpec((1, HIDDEN), const),
          pl.BlockSpec((1, HIDDEN), const),
      ],
      out_specs=pl.BlockSpec((R, HIDDEN), row),
      out_shape=jax.ShapeDtypeStruct((N, HIDDEN), jnp.float32),
  )(node_data, padded_index, etype_ids, fpar, tpar, femb, temb,
    W_node, b_node, order_table, etype_table,
    W1, b1, W2, b2, W3, b3, ln_gamma, ln_beta)


def kernel(node_data, padded_index, etype_ids, func_ids, token_ids,
           W_node, b_node, order_table, etype_table, func_table, token_table,
           W1, b1, W2, b2, W3, b3, ln_gamma, ln_beta):
  fids = func_ids.reshape(N)
  tids = token_ids.reshape(N)
  ttab_rm = _transpose(token_table.T)
  femb, temb = _sc_gather(
      fids >> 1, tids >> 1,
      func_table.reshape(func_table.shape[0] // 2, DW),
      ttab_rm.reshape(ttab_rm.shape[0] // 2, DW))
  out = _mlp(node_data.reshape(N, 4), padded_index.reshape(N, 2),
             etype_ids.reshape(N, 1), (fids & 1).reshape(N, 1),
             (tids & 1).reshape(N, 1), femb, temb,
             W_node, b_node.reshape(1, DH), order_table, etype_table,
             W1, b1.reshape(1, H1), W2, b2.reshape(1, HIDDEN),
             W3, b3.reshape(1, HIDDEN), ln_gamma.reshape(1, HIDDEN),
             ln_beta.reshape(1, HIDDEN))
  return out.reshape(B, T, HIDDEN)


# speedup vs baseline: 1.1689x; 1.0032x over previous
import functools

import jax
import jax.numpy as jnp
from jax import lax
from jax.experimental import pallas as pl
from jax.experimental.pallas import tpu as pltpu
from jax.experimental.pallas import tpu_sc as plsc

B, T = 4, 4096
N = B * T
DH = 64
DW = 2 * DH
HIDDEN = 1024
H1 = HIDDEN // 2
EPS = 1e-05

NC, NS = 2, 16
NW = NC * NS
BPW = N // NW
CH = 128
NCH = BPW // CH
NPASS = 2
SUB = BPW // NPASS
NCH_SUB = SUB // CH

R = 1024
NBLK = N // R

TC_COLS = 16384


def _transpose_body(in_ref, out_ref):
  x = in_ref[...]
  r = lax.broadcasted_iota(jnp.int32, (DH, DH), 0)
  c = lax.broadcasted_iota(jnp.int32, (DH, DH), 1)
  eye = (r == c).astype(jnp.float32)
  out_ref[...] = lax.dot_general(
      x, eye, dimension_numbers=(((0,), (0,)), ((), ())),
      preferred_element_type=jnp.float32)


@jax.jit
def _transpose(tab_t):
  V = tab_t.shape[1]
  grid = (V + TC_COLS - 1) // TC_COLS
  return pl.pallas_call(
      _transpose_body,
      grid=(grid,),
      in_specs=[pl.BlockSpec((DH, TC_COLS), lambda i: (0, i))],
      out_specs=pl.BlockSpec((TC_COLS, DH), lambda i: (i, 0)),
      out_shape=jax.ShapeDtypeStruct((V, DH), jnp.float32),
  )(tab_t)


def _sc_gather_body(fids_hbm, tids_hbm, ftab_hbm, ttab_hbm,
                    fout_hbm, tout_hbm,
                    fidx_v, tidx_v, frows_v, trows_v, fsem, tsem):
  wid = lax.axis_index("s") * NC + lax.axis_index("c")
  base = wid * BPW
  pltpu.sync_copy(fids_hbm.at[pl.ds(wid * NCH, NCH)], fidx_v)
  pltpu.sync_copy(tids_hbm.at[pl.ds(wid * NCH, NCH)], tidx_v)
  for p in range(NPASS):
    handles = []
    for j in range(NCH_SUB):
      c = p * NCH_SUB + j
      handles.append(pltpu.async_copy(
          ftab_hbm.at[fidx_v.at[c]], frows_v.at[pl.ds(j * CH, CH)], fsem))
      handles.append(pltpu.async_copy(
          ttab_hbm.at[tidx_v.at[c]], trows_v.at[pl.ds(j * CH, CH)], tsem))
    for h in handles:
      h.wait()
    pltpu.sync_copy(frows_v, fout_hbm.at[pl.ds(base + p * SUB, SUB)])
    pltpu.sync_copy(trows_v, tout_hbm.at[pl.ds(base + p * SUB, SUB)])


@jax.jit
def _sc_gather(fidx, tidx, func_table2, token_table2):
  mesh = plsc.VectorSubcoreMesh(
      core_axis_name="c", subcore_axis_name="s",
      num_cores=NC, num_subcores=NS)
  return pl.kernel(
      _sc_gather_body,
      out_type=[
          jax.ShapeDtypeStruct((N, DW), jnp.float32),
          jax.ShapeDtypeStruct((N, DW), jnp.float32),
      ],
      mesh=mesh,
      scratch_types=[
          pltpu.VMEM((NCH, CH), jnp.int32),
          pltpu.VMEM((NCH, CH), jnp.int32),
          pltpu.VMEM((SUB, DW), jnp.float32),
          pltpu.VMEM((SUB, DW), jnp.float32),
          pltpu.SemaphoreType.DMA,
          pltpu.SemaphoreType.DMA,
      ],
  )(fidx.reshape(N // CH, CH), tidx.reshape(N // CH, CH),
    func_table2, token_table2)


def _mlp_body(node_ref, pidx_ref, etype_ref, fpar_ref, tpar_ref,
              femb_ref, temb_ref,
              wn_ref, bn_ref, ot_ref, et_ref,
              w1_ref, b1_ref, w2_ref, b2_ref, w3_ref, b3_ref,
              g_ref, beta_ref, out_ref):
  f32 = jnp.float32
  nf = jnp.dot(node_ref[...], wn_ref[...], preferred_element_type=f32)
  nf = nf + bn_ref[...]
  o = (pidx_ref[:, 0:1] == pidx_ref[:, 1:2]).astype(f32)
  oe = ot_ref[0:1, :] * (1.0 - o) + ot_ref[1:2, :] * o
  eids = etype_ref[...]
  eoh = (eids == lax.broadcasted_iota(jnp.int32, (1, 4), 1)).astype(f32)
  ee = jnp.dot(eoh, et_ref[...], preferred_element_type=f32)
  fp = fpar_ref[...] > 0
  fe = jnp.where(fp, femb_ref[:, DH:DW], femb_ref[:, 0:DH])
  tp = tpar_ref[...] > 0
  te = jnp.where(tp, temb_ref[:, DH:DW], temb_ref[:, 0:DH])
  acc = jnp.dot(nf, w1_ref[0:DH, :], preferred_element_type=f32)
  acc += jnp.dot(oe, w1_ref[DH:2 * DH, :], preferred_element_type=f32)
  acc += jnp.dot(ee, w1_ref[2 * DH:3 * DH, :], preferred_element_type=f32)
  acc += jnp.dot(fe, w1_ref[3 * DH:4 * DH, :], preferred_element_type=f32)
  acc += jnp.dot(te, w1_ref[4 * DH:5 * DH, :], preferred_element_type=f32)
  acc += b1_ref[...]
  h = jnp.where(acc > 0, acc, 0.01 * acc)
  h = jnp.dot(h, w2_ref[...], preferred_element_type=f32) + b2_ref[...]
  h = jnp.where(h > 0, h, 0.01 * h)
  h = jnp.dot(h, w3_ref[...], preferred_element_type=f32) + b3_ref[...]
  mu = jnp.mean(h, axis=-1, keepdims=True)
  d = h - mu
  var = jnp.mean(d * d, axis=-1, keepdims=True)
  out_ref[...] = d * lax.rsqrt(var + EPS) * g_ref[...] + beta_ref[...]


@jax.jit
def _mlp(node_data, padded_index, etype_ids, fpar, tpar, femb, temb,
         W_node, b_node, order_table, etype_table,
         W1, b1, W2, b2, W3, b3, ln_gamma, ln_beta):
  row = lambda i: (i, 0)
  const = lambda i: (0, 0)
  return pl.pallas_call(
      _mlp_body,
      grid=(NBLK,),
      in_specs=[
          pl.BlockSpec((R, 4), row),
          pl.BlockSpec((R, 2), row),
          pl.BlockSpec((R, 1), row),
          pl.BlockSpec((R, 1), row),
          pl.BlockSpec((R, 1), row),
          pl.BlockSpec((R, DW), row),
          pl.BlockSpec((R, DW), row),
          pl.BlockSpec((4, DH), const),
          pl.BlockSpec((1, DH), const),
          pl.BlockSpec((2, DH), const),
          pl.BlockSpec((4, DH), const),
          pl.BlockSpec((5 * DH, H1), const),
          pl.BlockSpec((1, H1), const),
          pl.BlockSpec((H1, HIDDEN), const),
          pl.BlockSpec((1, HIDDEN), const),
          pl.BlockSpec((HIDDEN, HIDDEN), const),
          pl.BlockSpec((1, HIDDEN), const),
          pl.BlockSpec((1, HIDDEN), const),
          pl.BlockSpec((1, HIDDEN), const),
      ],
      out_specs=pl.BlockSpec((R, HIDDEN), row),
      out_shape=jax.ShapeDtypeStruct((N, HIDDEN), jnp.float32),
  )(node_data, padded_index, etype_ids, fpar, tpar, femb, temb,
    W_node, b_node, order_table, etype_table,
    W1, b1, W2, b2, W3, b3, ln_gamma, ln_beta)


def kernel(node_data, padded_index, etype_ids, func_ids, token_ids,
           W_node, b_node, order_table, etype_table, func_table, token_table,
           W1, b1, W2, b2, W3, b3, ln_gamma, ln_beta):
  fids = func_ids.reshape(N)
  tids = token_ids.reshape(N)
  ttab_rm = _transpose(token_table.T)
  femb, temb = _sc_gather(
      fids >> 1, tids >> 1,
      func_table.reshape(func_table.shape[0] // 2, DW),
      ttab_rm.reshape(ttab_rm.shape[0] // 2, DW))
  out = _mlp(node_data.reshape(N, 4), padded_index.reshape(N, 2),
             etype_ids.reshape(N, 1), (fids & 1).reshape(N, 1),
             (tids & 1).reshape(N, 1), femb, temb,
             W_node, b_node.reshape(1, DH), order_table, etype_table,
             W1, b1.reshape(1, H1), W2, b2.reshape(1, HIDDEN),
             W3, b3.reshape(1, HIDDEN), ln_gamma.reshape(1, HIDDEN),
             ln_beta.reshape(1, HIDDEN))
  return out.reshape(B, T, HIDDEN)
